# baseline (speedup 1.0000x reference)
import functools

import jax
import jax.numpy as jnp
from jax import lax
from jax.experimental import pallas as pl
from jax.experimental.pallas import tpu as pltpu
from jax.experimental.pallas import tpu_sc as plsc

NC, NS = 2, 16
NW = NC * NS
B, S, D = 16384, 20, 50
G = 16
TROW = 128
RPT = 2
DPAD = 64
TOK_PER_W = B * S // NW
SENT_PER_W = B // NW
ROWS_PER_SENT = S * RPT
NBUF = 2
INV_SQRT_S = float(1.0 / (S ** 0.5))

_mesh = plsc.VectorSubcoreMesh(
    core_axis_name="c", subcore_axis_name="s", num_cores=NC, num_subcores=NS)


@functools.partial(
    pl.kernel,
    out_type=jax.ShapeDtypeStruct((B, DPAD), jnp.float32),
    mesh=_mesh,
    scratch_types=[
        pltpu.VMEM((TOK_PER_W + G,), jnp.int32),
        pltpu.VMEM((SENT_PER_W, ROWS_PER_SENT), jnp.int32),
        pltpu.VMEM((ROWS_PER_SENT, TROW), jnp.float32),
        pltpu.VMEM((ROWS_PER_SENT, TROW), jnp.float32),
        pltpu.VMEM((SENT_PER_W, DPAD), jnp.float32),
        pltpu.SemaphoreType.DMA,
        pltpu.SemaphoreType.DMA,
    ],
    compiler_params=pltpu.CompilerParams(use_tc_tiling_on_sc=False,
                                         needs_layout_passes=False),
)
def _sc_embed(x_hbm, tview_hbm, out_hbm, ids_v, idx_v, rows0, rows1,
              sent_v, sem0, sem1):
    wid = lax.axis_index("s") * NC + lax.axis_index("c")
    iota = lax.iota(jnp.int32, G)
    mlow4 = iota < (S - G)

    pltpu.sync_copy(x_hbm.at[wid], ids_v.at[pl.ds(0, TOK_PER_W)])

    def genbody(s, carry):
        v0 = ids_v[pl.ds(s * S, G)]
        v1 = ids_v[pl.ds(s * S + G, G)]
        g0a = (v0 * (D // 2)) >> 6
        g0b = (v1 * (D // 2)) >> 6
        rmax = tview_hbm.shape[0] - 1
        srow = jnp.zeros((G,), jnp.int32) + s
        ca = iota * RPT
        cb = ca + G * RPT
        plsc.store_scatter(idx_v, [srow, ca], g0a)
        plsc.store_scatter(idx_v, [srow, ca + 1], jnp.minimum(g0a + 1, rmax))
        plsc.store_scatter(idx_v, [srow, cb], g0b, mask=mlow4)
        plsc.store_scatter(idx_v, [srow, cb + 1],
                           jnp.minimum(g0b + 1, rmax), mask=mlow4)
        return carry

    lax.fori_loop(0, SENT_PER_W, genbody, 0)

    rows = (rows0, rows1)
    sems = (sem0, sem1)

    def copy(s, b):
        return pltpu.make_async_copy(
            tview_hbm.at[idx_v.at[s]], rows[b], sems[b])

    for b in range(NBUF):
        copy(b, b).start()

    zero = jnp.zeros((G,), jnp.float32)

    def outer(o, carry):
        for b in range(NBUF):
            s = o * NBUF + b
            copy(s, b).wait()
            pa = (ids_v[pl.ds(s * S, G)] * D) & (TROW - 1)
            pb = (ids_v[pl.ds(s * S + (S - G), G)] * D) & (TROW - 1)
            accs = [zero, zero, zero, zero]
            for t in range(S):
                p = pa[t] if t < G else pb[t - (S - G)]
                colv = p + iota
                for c4 in range(4):
                    cc = colv + c4 * G
                    rv = (cc >> 7) + (t * RPT)
                    accs[c4] = accs[c4] + plsc.load_gather(
                        rows[b], [rv, cc & (TROW - 1)])
            accs = [a * INV_SQRT_S for a in accs]
            accs[3] = jnp.where(iota < (D - 3 * G), accs[3], 0.0)
            sent_v[s, pl.ds(0, G)] = accs[0]
            sent_v[s, pl.ds(G, G)] = accs[1]
            sent_v[s, pl.ds(2 * G, G)] = accs[2]
            sent_v[s, pl.ds(3 * G, G)] = accs[3]
            nxt = s + NBUF
            @pl.when(nxt < SENT_PER_W)
            def _():
                copy(nxt, b).start()
        return carry

    lax.fori_loop(0, SENT_PER_W // NBUF, outer, 0)
    pltpu.sync_copy(sent_v, out_hbm.at[pl.ds(wid * SENT_PER_W, SENT_PER_W)])


def _mlp_body(sent_ref, w1_ref, b1_ref, w2_ref, b2_ref, out_ref):
    s = sent_ref[...]
    h = jnp.dot(s, w1_ref[...], preferred_element_type=jnp.float32)
    h = jnp.maximum(h + b1_ref[...], 0.0)
    out_ref[...] = jnp.sum(h * w2_ref[...], axis=1, keepdims=True) + b2_ref[...]


def kernel(x, table, W1, b1, W2, b2):
    x2 = x.reshape(NW, TOK_PER_W).astype(jnp.int32)
    tview = table.reshape((table.shape[0] * D) // TROW, TROW)
    sent = _sc_embed(x2, tview)

    w1p = jnp.zeros((DPAD, 16), jnp.float32).at[:D].set(W1.astype(jnp.float32))
    BLK = 2048
    out = pl.pallas_call(
        _mlp_body,
        grid=(B // BLK,),
        in_specs=[
            pl.BlockSpec((BLK, DPAD), lambda i: (i, 0)),
            pl.BlockSpec((DPAD, 16), lambda i: (0, 0)),
            pl.BlockSpec((1, 16), lambda i: (0, 0)),
            pl.BlockSpec((1, 16), lambda i: (0, 0)),
            pl.BlockSpec((1, 1), lambda i: (0, 0)),
        ],
        out_specs=pl.BlockSpec((BLK, 1), lambda i: (i, 0)),
        out_shape=jax.ShapeDtypeStruct((B, 1), jnp.float32),
    )(sent, w1p, b1.reshape(1, 16).astype(jnp.float32),
      W2.reshape(1, 16).astype(jnp.float32),
      b2.reshape(1, 1).astype(jnp.float32))
    return out


# speedup vs baseline: 1.3839x; 1.3839x over previous
import functools

import jax
import jax.numpy as jnp
from jax import lax
from jax.experimental import pallas as pl
from jax.experimental.pallas import tpu as pltpu
from jax.experimental.pallas import tpu_sc as plsc

NC, NS = 2, 16
NW = NC * NS
B, S, D = 16384, 20, 50
G = 16
DPAD = 64
SENT_PER_CHUNK = 4
TOK_PER_CHUNK = SENT_PER_CHUNK * S
SENT_PER_W = B // NW
CHUNKS_PER_W = SENT_PER_W // SENT_PER_CHUNK
NBUF = 2
INV_SQRT_S = float(1.0 / (S ** 0.5))

_mesh = plsc.VectorSubcoreMesh(
    core_axis_name="c", subcore_axis_name="s", num_cores=NC, num_subcores=NS)


@functools.partial(
    pl.kernel,
    out_type=jax.ShapeDtypeStruct((B, DPAD), jnp.float32),
    mesh=_mesh,
    scratch_types=[
        pltpu.VMEM((CHUNKS_PER_W, TOK_PER_CHUNK), jnp.int32),
        pltpu.VMEM((TOK_PER_CHUNK, DPAD), jnp.float32),
        pltpu.VMEM((TOK_PER_CHUNK, DPAD), jnp.float32),
        pltpu.VMEM((SENT_PER_W, DPAD), jnp.float32),
        pltpu.SemaphoreType.DMA,
        pltpu.SemaphoreType.DMA,
    ],
    compiler_params=pltpu.CompilerParams(use_tc_tiling_on_sc=False,
                                         needs_layout_passes=False),
)
def _sc_embed(x_hbm, tview_hbm, out_hbm, ids_v, rows0, rows1,
              sent_v, sem0, sem1):
    wid = lax.axis_index("s") * NC + lax.axis_index("c")

    pltpu.sync_copy(x_hbm.at[wid], ids_v)

    rows = (rows0, rows1)
    sems = (sem0, sem1)

    def copy(i, b):
        return pltpu.make_async_copy(
            tview_hbm.at[ids_v.at[i]], rows[b], sems[b])

    for b in range(NBUF):
        copy(b, b).start()

    def outer(o, carry):
        for b in range(NBUF):
            i = o * NBUF + b
            copy(i, b).wait()
            for t in range(SENT_PER_CHUNK):
                row0 = t * S
                for c in range(4):
                    acc = rows[b][row0, pl.ds(c * G, G)]
                    for s2 in range(1, S):
                        acc = acc + rows[b][row0 + s2, pl.ds(c * G, G)]
                    sent_v[i * SENT_PER_CHUNK + t, pl.ds(c * G, G)] = (
                        acc * INV_SQRT_S)
            nxt = i + NBUF
            @pl.when(nxt < CHUNKS_PER_W)
            def _():
                copy(nxt, b).start()
        return carry

    lax.fori_loop(0, CHUNKS_PER_W // NBUF, outer, 0)
    pltpu.sync_copy(sent_v, out_hbm.at[pl.ds(wid * SENT_PER_W, SENT_PER_W)])


def _repack_body(in_ref, out_ref):
    x = in_ref[...]
    out_ref[...] = jnp.concatenate(
        [x, jnp.zeros((x.shape[0], DPAD - D), jnp.float32)], axis=1)


def _repack(table):
    rblk = 4000
    grid = -(-table.shape[0] // rblk)
    return pl.pallas_call(
        _repack_body,
        grid=(grid,),
        in_specs=[pl.BlockSpec((rblk, D), lambda i: (i, 0))],
        out_specs=pl.BlockSpec((rblk, DPAD), lambda i: (i, 0)),
        out_shape=jax.ShapeDtypeStruct((grid * rblk, DPAD), jnp.float32),
    )(table)


def _mlp_body(sent_ref, w1_ref, b1_ref, w2_ref, b2_ref, out_ref):
    s = sent_ref[...]
    h = jnp.dot(s, w1_ref[...], preferred_element_type=jnp.float32)
    h = jnp.maximum(h + b1_ref[...], 0.0)
    out_ref[...] = jnp.sum(h * w2_ref[...], axis=1, keepdims=True) + b2_ref[...]


def kernel(x, table, W1, b1, W2, b2):
    x3 = x.reshape(NW, CHUNKS_PER_W, TOK_PER_CHUNK).astype(jnp.int32)
    tview = _repack(table)
    sent = _sc_embed(x3, tview)

    w1p = jnp.zeros((DPAD, 16), jnp.float32).at[:D].set(W1.astype(jnp.float32))
    BLK = 2048
    out = pl.pallas_call(
        _mlp_body,
        grid=(B // BLK,),
        in_specs=[
            pl.BlockSpec((BLK, DPAD), lambda i: (i, 0)),
            pl.BlockSpec((DPAD, 16), lambda i: (0, 0)),
            pl.BlockSpec((1, 16), lambda i: (0, 0)),
            pl.BlockSpec((1, 16), lambda i: (0, 0)),
            pl.BlockSpec((1, 1), lambda i: (0, 0)),
        ],
        out_specs=pl.BlockSpec((BLK, 1), lambda i: (i, 0)),
        out_shape=jax.ShapeDtypeStruct((B, 1), jnp.float32),
    )(sent, w1p, b1.reshape(1, 16).astype(jnp.float32),
      W2.reshape(1, 16).astype(jnp.float32),
      b2.reshape(1, 1).astype(jnp.float32))
    return out
